# baseline (speedup 1.0000x reference)
import jax
import jax.numpy as jnp
from jax import lax
from jax.experimental import pallas as pl
from jax.experimental.pallas import tpu as pltpu
from jax.experimental.pallas import tpu_sc as plsc

EMBED = 64
NC, NS = 2, 16
NW = NC * NS
SB, BBD, SOD, BL = 25, 32, 8, 128
JJ = 4
UNITS = SB * BBD * (SOD // JJ)
UPW = UNITS // NW


def _body(table_hbm, seqv_hbm, out_hbm, idx_v, rows_v, out_f, *sems):
    idx_sems = sems[0:2]
    gat_sems = sems[2:4]
    out_sems = sems[4:8]
    wid = lax.axis_index("s") * NC + lax.axis_index("c")

    lane = jnp.arange(16, dtype=jnp.int32)
    base_g = [(lane + 16 * g) * BL for g in range(4)]

    def unit_coords(k):
        u = wid + NW * k
        sp = u % 2
        bb = (u // 2) % BBD
        sb = u // (2 * BBD)
        return sb, bb, sp

    def idx_src(k):
        sb, bb, sp = unit_coords(k)
        return seqv_hbm.at[sb, bb, pl.ds(JJ * sp, JJ), :]

    def fire_idx(k, b):
        pltpu.async_copy(idx_src(k), idx_v.at[b], idx_sems[b])

    def fire_gathers(k, b):
        for jj in range(JJ):
            pltpu.async_copy(
                table_hbm.at[idx_v.at[b, jj]], rows_v.at[b, jj], gat_sems[b]
            )

    def drain_gathers(k, b):
        for jj in range(JJ):
            pltpu.make_async_copy(
                table_hbm.at[idx_v.at[b, jj]], rows_v.at[b, jj], gat_sems[b]
            ).wait()

    def out_copies(k, j):
        sb, bb, sp = unit_coords(k)
        s = 8 * sb + JJ * sp + j
        return [
            (out_f.at[j, pl.ds(eb * 1024, 1024)], out_hbm.at[s, eb, bb])
            for eb in range(8)
        ]

    fire_idx(0, 0)
    fire_idx(1, 1)
    pltpu.make_async_copy(idx_src(0), idx_v.at[0], idx_sems[0]).wait()
    fire_gathers(0, 0)

    @pl.loop(0, UPW, step=2)
    def _step(i):
        for b in range(2):
            k = i + b

            drain_gathers(k, b)

            @pl.when(k + 2 < UPW)
            def _():
                fire_idx(k + 2, b)

            @pl.when(k + 1 < UPW)
            def _():
                pltpu.make_async_copy(
                    idx_src(k + 1), idx_v.at[1 - b], idx_sems[1 - b]
                ).wait()
                fire_gathers(k + 1, 1 - b)

            for j in range(JJ):
                @pl.when(k >= 1)
                def _():
                    for src, dst in out_copies(k, j):
                        pltpu.make_async_copy(src, dst, out_sems[j]).wait()

                @pl.loop(0, BL)
                def _tok(bl):
                    col = jnp.full((16,), bl, dtype=jnp.int32)
                    for g in range(4):
                        x = rows_v[b, j, bl, pl.ds(16 * g, 16)]
                        plsc.store_scatter(out_f.at[j], [base_g[g] + col], x)

                for src, dst in out_copies(k, j):
                    pltpu.async_copy(src, dst, out_sems[j])

    for j in range(JJ):
        for src, dst in out_copies(UPW - 1, j):
            pltpu.make_async_copy(src, dst, out_sems[j]).wait()


def kernel(seq, table):
    seqv = seq.T.reshape(SB, SOD, BBD, BL).transpose(0, 2, 1, 3)
    mesh = plsc.VectorSubcoreMesh(core_axis_name="c", subcore_axis_name="s")
    run = pl.kernel(
        _body,
        out_type=jax.ShapeDtypeStruct((200, 8, BBD, 1024), jnp.float32),
        mesh=mesh,
        scratch_types=[
            pltpu.VMEM((2, JJ, BL), jnp.int32),
            pltpu.VMEM((2, JJ, BL, EMBED), jnp.float32),
            pltpu.VMEM((JJ, 8 * 1024), jnp.float32),
        ] + [pltpu.SemaphoreType.DMA] * 8,
        compiler_params=pltpu.CompilerParams(
            use_tc_tiling_on_sc=False, needs_layout_passes=False
        ),
    )
    out5 = run(table, seqv)
    out5 = out5.reshape(200, 8, BBD, 8, BL)
    return out5.transpose(2, 4, 0, 1, 3).reshape(4096, 200, EMBED)


# speedup vs baseline: 1.2933x; 1.2933x over previous
import jax
import jax.numpy as jnp
from jax import lax
from jax.experimental import pallas as pl
from jax.experimental.pallas import tpu as pltpu
from jax.experimental.pallas import tpu_sc as plsc

EMBED = 64
NC, NS = 2, 16
NW = NC * NS
CHUNK = 512
NBUF = 2


def _gather_body(table_hbm, idx_hbm, out_hbm, idx_v, rows_v, *sems):
    idx_sems = sems[0:NBUF]
    gat_sems = sems[NBUF:2 * NBUF]
    out_sems = sems[2 * NBUF:3 * NBUF]
    wid = lax.axis_index("s") * NC + lax.axis_index("c")
    n = idx_hbm.shape[0]
    npw = n // NW
    nchunks = npw // CHUNK
    base0 = wid * npw

    def idx_src(ci):
        return idx_hbm.at[pl.ds(base0 + ci * CHUNK, CHUNK)]

    def out_dst(ci):
        return out_hbm.at[pl.ds(base0 + ci * CHUNK, CHUNK)]

    for b in range(NBUF):
        pltpu.async_copy(idx_src(b), idx_v.at[b], idx_sems[b])

    @pl.loop(0, nchunks, step=NBUF)
    def _step(i):
        for b in range(NBUF):
            ci = i + b

            @pl.when(ci >= NBUF)
            def _():
                pltpu.make_async_copy(rows_v.at[b], out_dst(ci), out_sems[b]).wait()

            pltpu.make_async_copy(idx_src(ci), idx_v.at[b], idx_sems[b]).wait()

            pltpu.async_copy(
                table_hbm.at[idx_v.at[b]], rows_v.at[b], gat_sems[b]
            ).wait()

            @pl.when(ci + NBUF < nchunks)
            def _():
                pltpu.async_copy(idx_src(ci + NBUF), idx_v.at[b], idx_sems[b])

            pltpu.async_copy(rows_v.at[b], out_dst(ci), out_sems[b])

    for b in range(NBUF):
        pltpu.make_async_copy(
            rows_v.at[b], out_dst(nchunks - NBUF + b), out_sems[b]
        ).wait()


def kernel(seq, table):
    b, s = seq.shape
    n = b * s
    idx = seq.reshape(n)
    mesh = plsc.VectorSubcoreMesh(core_axis_name="c", subcore_axis_name="s")
    run = pl.kernel(
        _gather_body,
        out_type=jax.ShapeDtypeStruct((n, EMBED), jnp.float32),
        mesh=mesh,
        scratch_types=[
            pltpu.VMEM((NBUF, CHUNK), jnp.int32),
            pltpu.VMEM((NBUF, CHUNK, EMBED), jnp.float32),
        ] + [pltpu.SemaphoreType.DMA] * (3 * NBUF),
        compiler_params=pltpu.CompilerParams(use_tc_tiling_on_sc=False),
    )
    out = run(table, idx)
    return out.reshape(b, s, EMBED)
